# baseline (speedup 1.0000x reference)
import functools

import jax
import jax.numpy as jnp
from jax.experimental import pallas as pl
from jax.experimental.pallas import tpu as pltpu

_MIB = 1024 * 1024


def _se_pipe(x_hbm, w1_ref, b1_ref, w2_ref, b2_ref, o_hbm,
             x_buf, o_buf, in_sem, out_sem, *, bt, n_steps):
    def dma_in(slot, step):
        pltpu.make_async_copy(x_hbm.at[pl.ds(step * bt, bt)],
                              x_buf.at[slot], in_sem.at[slot]).start()

    def wait_in(slot):
        pltpu.make_async_copy(x_buf.at[slot], x_buf.at[slot],
                              in_sem.at[slot]).wait()

    def dma_out(slot, step):
        pltpu.make_async_copy(o_buf.at[slot],
                              o_hbm.at[pl.ds(step * bt, bt)],
                              out_sem.at[slot]).start()

    def wait_out(slot):
        pltpu.make_async_copy(o_buf.at[slot], o_buf.at[slot],
                              out_sem.at[slot]).wait()

    dma_in(0, 0)

    def body(step, _):
        cur = jax.lax.rem(step, 2)
        nxt = jax.lax.rem(step + 1, 2)

        @pl.when(step + 1 < n_steps)
        def _():
            dma_in(nxt, step + 1)

        wait_in(cur)

        @pl.when(step >= 2)
        def _():
            wait_out(cur)

        s = jnp.sum(x_buf[cur], axis=2, dtype=jnp.float32)
        h = jnp.dot(s, w1_ref[...], preferred_element_type=jnp.float32)
        h = jnp.maximum(h + b1_ref[...], 0.0)
        g = jnp.dot(h, w2_ref[...], preferred_element_type=jnp.float32)
        g = jax.nn.sigmoid(g + b2_ref[...])
        o_buf[cur] = x_buf[cur] * g[:, :, None]
        dma_out(cur, step)
        return ()

    jax.lax.fori_loop(0, n_steps, body, ())
    wait_out(jax.lax.rem(n_steps - 2, 2))
    wait_out(jax.lax.rem(n_steps - 1, 2))


@jax.jit
def kernel(x, w1, b1, w2, b2):
    B, C, H, W = x.shape
    Cr = w1.shape[0]
    HW = H * W
    f32 = jnp.float32

    x3 = x.reshape(B, C, HW)
    w1t = jnp.transpose(w1).astype(f32) * (1.0 / HW)
    w2t = jnp.transpose(w2).astype(f32)
    b1r = b1.reshape(1, Cr).astype(f32)
    b2r = b2.reshape(1, C).astype(f32)

    bt = 8
    n_steps = B // bt
    buf_bytes = 2 * 2 * bt * C * HW * jnp.dtype(x.dtype).itemsize

    out = pl.pallas_call(
        functools.partial(_se_pipe, bt=bt, n_steps=n_steps),
        out_shape=jax.ShapeDtypeStruct((B, C, HW), x.dtype),
        in_specs=[
            pl.BlockSpec(memory_space=pl.ANY),
            pl.BlockSpec(memory_space=pltpu.VMEM),
            pl.BlockSpec(memory_space=pltpu.VMEM),
            pl.BlockSpec(memory_space=pltpu.VMEM),
            pl.BlockSpec(memory_space=pltpu.VMEM),
        ],
        out_specs=pl.BlockSpec(memory_space=pl.ANY),
        scratch_shapes=[
            pltpu.VMEM((2, bt, C, HW), x.dtype),
            pltpu.VMEM((2, bt, C, HW), x.dtype),
            pltpu.SemaphoreType.DMA((2,)),
            pltpu.SemaphoreType.DMA((2,)),
        ],
        compiler_params=pltpu.CompilerParams(
            vmem_limit_bytes=buf_bytes + 8 * _MIB,
        ),
    )(x3, w1t, b1r, w2t, b2r)
    return out.reshape(B, C, H, W)
